# baseline (speedup 1.0000x reference)
import functools

import jax
import jax.numpy as jnp
from jax import lax
from jax.experimental import pallas as pl
from jax.experimental.pallas import tpu as pltpu
from jax.experimental.pallas import tpu_sc as plsc

D = 64
T = 200
B = 4096
NC, NS = 2, 16
NW = NC * NS
BB = 128
NBB = B // BB
NBLK = T * NBB
BLK_PER_W = NBLK // NW

_mesh = plsc.VectorSubcoreMesh(core_axis_name="c", subcore_axis_name="s")


@functools.partial(
    pl.kernel,
    out_type=jax.ShapeDtypeStruct((T, 8, NBB, 8, BB), jnp.float32),
    mesh=_mesh,
    scratch_types=[
        [pltpu.VMEM((BB,), jnp.int32)] * 2,
        [pltpu.VMEM((BB, D), jnp.float32)] * 2,
        [pltpu.VMEM((8, 8, BB), jnp.float32)] * 2,
        pltpu.VMEM((T * D,), jnp.float32),
        [pltpu.SemaphoreType.DMA] * 2,
        [pltpu.SemaphoreType.DMA] * 2,
        [pltpu.SemaphoreType.DMA] * 2,
    ],
    compiler_params=pltpu.CompilerParams(
        use_tc_tiling_on_sc=False, needs_layout_passes=False),
)
def _embed(xt_hbm, tok_hbm, pos_hbm, out_hbm,
           idx_v, gbuf, tbuf, pos_v, isem, gsem, ssem):
    wid = lax.axis_index("s") * NC + lax.axis_index("c")
    pltpu.sync_copy(pos_hbm, pos_v)
    base = wid * BLK_PER_W
    lane = lax.broadcasted_iota(jnp.int32, (16,), 0)
    rowv = [lane + bq * 16 for bq in range(BB // 16)]

    def tb(i):
        bid = base + i
        return bid // NBB, bid % NBB

    def start_idx(i, p):
        t, bc = tb(i)
        pltpu.async_copy(
            xt_hbm.at[t, pl.ds(bc * BB, BB)], idx_v[p], isem[p])

    def start_gather(i, p):
        pltpu.make_async_copy(
            xt_hbm.at[0, pl.ds(0, BB)], idx_v[p], isem[p]).wait()
        pltpu.async_copy(tok_hbm.at[idx_v[p]], gbuf[p], gsem[p])

    def start_store(i, p):
        t, bc = tb(i)
        for dr in range(8):
            pltpu.async_copy(
                tbuf[p].at[dr], out_hbm.at[t, dr, bc], ssem[p])

    def drain_store(p):
        pltpu.make_async_copy(
            out_hbm.at[0, 0].at[pl.ds(0, 8)], tbuf[p], ssem[p]).wait()

    start_idx(0, 0)
    start_idx(1, 1)
    start_gather(0, 0)
    start_gather(1, 1)

    @pl.loop(0, BLK_PER_W, step=2)
    def _(i0):
        for p in range(2):
            i = i0 + p
            t, _bc = tb(i)
            pltpu.make_async_copy(
                tok_hbm.at[pl.ds(0, BB)], gbuf[p], gsem[p]).wait()
            @pl.when(i + 2 < BLK_PER_W)
            def _():
                start_idx(i + 2, p)
            @pl.when(i >= 2)
            def _():
                drain_store(p)

            @pl.loop(0, 8)
            def _(dr):
                for di in range(8):
                    d = dr * 8 + di
                    cold = jnp.broadcast_to(d, (16,))
                    posd = plsc.load_gather(
                        pos_v, [jnp.broadcast_to(t * D + d, (16,))])
                    for bq in range(BB // 16):
                        vals = plsc.load_gather(gbuf[p], [rowv[bq], cold])
                        tbuf[p][dr, di, pl.ds(bq * 16, 16)] = vals + posd

            @pl.when(i + 2 < BLK_PER_W)
            def _():
                start_gather(i + 2, p)
            start_store(i, p)

    drain_store(0)
    drain_store(1)


def kernel(x, token_table, pos_table):
    xt = x.astype(jnp.int32).T
    pos = pos_table.reshape(-1)
    o = _embed(xt, token_table, pos)
    return o.transpose(2, 4, 0, 1, 3).reshape(B, T, D)


# speedup vs baseline: 1.7198x; 1.7198x over previous
import functools

import jax
import jax.numpy as jnp
from jax import lax
from jax.experimental import pallas as pl
from jax.experimental.pallas import tpu as pltpu
from jax.experimental.pallas import tpu_sc as plsc

D = 64
T = 200
B = 4096
NC, NS = 2, 16
NW = NC * NS
BB = 128
NBB = B // BB
NBLK = T * NBB
BLK_PER_W = NBLK // NW

_mesh = plsc.VectorSubcoreMesh(core_axis_name="c", subcore_axis_name="s")


@functools.partial(
    pl.kernel,
    out_type=jax.ShapeDtypeStruct((T, 8, NBB, 8, BB), jnp.float32),
    mesh=_mesh,
    scratch_types=[
        [pltpu.VMEM((BB,), jnp.int32)] * 2,
        [pltpu.VMEM((BB, D), jnp.float32)] * 2,
        [pltpu.VMEM((8, 8, BB), jnp.float32)] * 2,
        pltpu.VMEM((T * D,), jnp.float32),
        [pltpu.SemaphoreType.DMA] * 2,
        [pltpu.SemaphoreType.DMA] * 2,
        [pltpu.SemaphoreType.DMA] * 2,
    ],
    compiler_params=pltpu.CompilerParams(
        use_tc_tiling_on_sc=False, needs_layout_passes=False),
)
def _embed(xt_hbm, tok_hbm, pos_hbm, out_hbm,
           idx_v, gbuf, tbuf, pos_v, isem, gsem, ssem):
    wid = lax.axis_index("s") * NC + lax.axis_index("c")
    pltpu.sync_copy(pos_hbm, pos_v)
    base = wid * BLK_PER_W
    lane = lax.broadcasted_iota(jnp.int32, (16,), 0)
    rowv = [lane + bq * 16 for bq in range(BB // 16)]

    def tb(i):
        bid = base + i
        return bid // NBB, bid % NBB

    def start_idx(i, p):
        t, bc = tb(i)
        pltpu.async_copy(
            xt_hbm.at[t, pl.ds(bc * BB, BB)], idx_v[p], isem[p])

    def start_gather(i, p):
        pltpu.make_async_copy(
            xt_hbm.at[0, pl.ds(0, BB)], idx_v[p], isem[p]).wait()
        pltpu.async_copy(tok_hbm.at[idx_v[p]], gbuf[p], gsem[p])

    def start_store(i, p):
        t, bc = tb(i)
        for dr in range(8):
            pltpu.async_copy(
                tbuf[p].at[dr], out_hbm.at[t, dr, bc], ssem[p])

    def drain_store(p):
        pltpu.make_async_copy(
            out_hbm.at[0, 0].at[pl.ds(0, 8)], tbuf[p], ssem[p]).wait()

    start_idx(0, 0)
    start_idx(1, 1)
    start_gather(0, 0)
    start_gather(1, 1)

    @pl.loop(0, BLK_PER_W, step=2)
    def _(i0):
        for p in range(2):
            i = i0 + p
            t, _bc = tb(i)
            pltpu.make_async_copy(
                tok_hbm.at[pl.ds(0, BB)], gbuf[p], gsem[p]).wait()
            @pl.when(i + 2 < BLK_PER_W)
            def _():
                start_idx(i + 2, p)
            @pl.when(i >= 2)
            def _():
                drain_store(p)

            @pl.loop(0, D)
            def _(kk):
                dvec = ((lane + kk) & 15) + (kk & (D - 16))
                posv = plsc.load_gather(pos_v, [dvec + t * D])
                drv = lax.shift_right_logical(dvec, 3)
                div = dvec & 7
                for bq in range(BB // 16):
                    vals = plsc.load_gather(gbuf[p], [rowv[bq], dvec])
                    plsc.store_scatter(
                        tbuf[p], [drv, div, rowv[bq]], vals + posv)

            @pl.when(i + 2 < BLK_PER_W)
            def _():
                start_gather(i + 2, p)
            start_store(i, p)

    drain_store(0)
    drain_store(1)


def kernel(x, token_table, pos_table):
    xt = x.astype(jnp.int32).T
    pos = pos_table.reshape(-1)
    o = _embed(xt, token_table, pos)
    return o.transpose(2, 4, 0, 1, 3).reshape(B, T, D)


# speedup vs baseline: 1.7297x; 1.0057x over previous
import functools

import jax
import jax.numpy as jnp
from jax import lax
from jax.experimental import pallas as pl
from jax.experimental.pallas import tpu as pltpu
from jax.experimental.pallas import tpu_sc as plsc

D = 64
T = 200
B = 4096
NC, NS = 2, 16
NW = NC * NS
BB = 128
NBB = B // BB
NBLK = T * NBB
BLK_PER_W = NBLK // NW

_mesh = plsc.VectorSubcoreMesh(core_axis_name="c", subcore_axis_name="s")


@functools.partial(
    pl.kernel,
    out_type=jax.ShapeDtypeStruct((T, 8, NBB, 8, BB), jnp.float32),
    mesh=_mesh,
    scratch_types=[
        [pltpu.VMEM((BB,), jnp.int32)] * 2,
        [pltpu.VMEM((BB, D), jnp.float32)] * 2,
        [pltpu.VMEM((8, 8, BB), jnp.float32)] * 2,
        pltpu.VMEM((T * D,), jnp.float32),
        [pltpu.SemaphoreType.DMA] * 2,
        [pltpu.SemaphoreType.DMA] * 2,
        [pltpu.SemaphoreType.DMA] * 2,
    ],
    compiler_params=pltpu.CompilerParams(
        use_tc_tiling_on_sc=False, needs_layout_passes=False),
)
def _embed(xt_hbm, tok_hbm, pos_hbm, out_hbm,
           idx_v, gbuf, tbuf, pos_v, isem, gsem, ssem):
    wid = lax.axis_index("s") * NC + lax.axis_index("c")
    pltpu.sync_copy(pos_hbm, pos_v)
    base = wid * BLK_PER_W
    lane = lax.broadcasted_iota(jnp.int32, (16,), 0)
    rowv = [lane + bq * 16 for bq in range(BB // 16)]

    def tb(i):
        bid = base + i
        return bid // NBB, bid % NBB

    def start_idx(i, p):
        t, bc = tb(i)
        pltpu.async_copy(
            xt_hbm.at[t, pl.ds(bc * BB, BB)], idx_v[p], isem[p])

    def start_gather(i, p):
        pltpu.make_async_copy(
            xt_hbm.at[0, pl.ds(0, BB)], idx_v[p], isem[p]).wait()
        pltpu.async_copy(tok_hbm.at[idx_v[p]], gbuf[p], gsem[p])

    def start_store(i, p):
        t, bc = tb(i)
        pltpu.async_copy(tbuf[p], out_hbm.at[t, :, bc], ssem[p])

    def drain_store(p):
        pltpu.make_async_copy(
            out_hbm.at[0, 0].at[pl.ds(0, 8)], tbuf[p], ssem[p]).wait()

    start_idx(0, 0)
    start_idx(1, 1)
    start_gather(0, 0)
    start_gather(1, 1)

    @pl.loop(0, BLK_PER_W, step=2)
    def _(i0):
        for p in range(2):
            i = i0 + p
            t, _bc = tb(i)
            pltpu.make_async_copy(
                tok_hbm.at[pl.ds(0, BB)], gbuf[p], gsem[p]).wait()
            @pl.when(i + 2 < BLK_PER_W)
            def _():
                start_idx(i + 2, p)
            @pl.when(i >= 2)
            def _():
                drain_store(p)

            @pl.loop(0, D)
            def _(kk):
                dvec = ((lane + kk) & 15) + (kk & (D - 16))
                posv = plsc.load_gather(pos_v, [dvec + t * D])
                drv = lax.shift_right_logical(dvec, 3)
                div = dvec & 7
                for bq in range(BB // 16):
                    vals = plsc.load_gather(gbuf[p], [rowv[bq], dvec])
                    plsc.store_scatter(
                        tbuf[p], [drv, div, rowv[bq]], vals + posv)

            @pl.when(i + 2 < BLK_PER_W)
            def _():
                start_gather(i + 2, p)
            start_store(i, p)

    drain_store(0)
    drain_store(1)


def kernel(x, token_table, pos_table):
    xt = x.astype(jnp.int32).T
    pos = pos_table.reshape(-1)
    o = _embed(xt, token_table, pos)
    return o.transpose(2, 4, 0, 1, 3).reshape(B, T, D)
